# baseline (speedup 1.0000x reference)
import jax
import jax.numpy as jnp
from jax import lax
from jax.experimental import pallas as pl
from jax.experimental.pallas import tpu as pltpu
from jax.experimental.pallas import tpu_sc as plsc

N = 10000
E = 320000
D = 128
ED = 16
B = 64

NC, NS = 2, 16
HD = D // NC
EPT = E // NS
C = 80
NCHUNK = EPT // C
NPAD = 10240
RPT = NPAD // NS

NB = 5
NROUND = NCHUNK // NB


def _make_segsum(with_edge):
    mesh = plsc.VectorSubcoreMesh(core_axis_name="c", subcore_axis_name="s")
    scratch = [
        pltpu.VMEM((2, NB, C), jnp.int32),
        pltpu.VMEM((2, NB, C), jnp.int32),
        [pltpu.VMEM((C, HD), jnp.float32) for _ in range(NB)],
    ]
    if with_edge:
        scratch.append(
            [pltpu.VMEM((C // 2, D), jnp.float32) for _ in range(NB)])
    scratch += [
        pltpu.VMEM_SHARED((NPAD, HD), jnp.float32),
        pltpu.SemaphoreType.DMA((NB,)),
        pltpu.SemaphoreType.DMA((NB,)),
        pltpu.SemaphoreType.DMA,
    ]
    if with_edge:
        scratch.append(pltpu.SemaphoreType.DMA((NB,)))

    def body(table, srcs5, dsts4, *rest):
        if with_edge:
            (e_hbm, zeros, out, idx_s, idx_d, rows, ebuf, accum, gsem, ssem,
             isem, esem) = rest
        else:
            zeros, out, idx_s, idx_d, rows, accum, gsem, ssem, isem = rest
        c = lax.axis_index("c")
        s = lax.axis_index("s")
        col = c * HD
        pltpu.sync_copy(zeros.at[pl.ds(s * RPT, RPT)],
                        accum.at[pl.ds(s * RPT, RPT)])
        pltpu.sync_copy(srcs5.at[c, s, 0], idx_s.at[0])
        pltpu.sync_copy(dsts4.at[s, 0], idx_d.at[0])
        plsc.subcore_barrier()

        def gstart(chunk, slot, b):
            pltpu.async_copy(table.at[idx_s.at[slot, b]], rows[b],
                             gsem.at[b])
            if with_edge:
                pltpu.async_copy(
                    e_hbm.at[c, pl.ds((s * EPT + chunk * C) // 2, C // 2)],
                    ebuf[b], esem.at[b])

        def gwait(b):
            pltpu.make_async_copy(table.at[idx_s.at[0, 0]], rows[b],
                                  gsem.at[b]).wait()
            if with_edge:
                pltpu.make_async_copy(e_hbm.at[0, pl.ds(0, C // 2)],
                                      ebuf[b], esem.at[b]).wait()

        def compute(b):
            if with_edge:
                @pl.loop(0, C // 2, unroll=4)
                def _(j):
                    for p in range(2):
                        for jj in range(HD // 16):
                            sl = pl.ds(jj * 16, 16)
                            se = pl.ds(p * HD + jj * 16, 16)
                            rows[b][2 * j + p, sl] = jnp.maximum(
                                rows[b][2 * j + p, sl] + ebuf[b][j, se], 0.0)

        def sstart(slot, b):
            pltpu.async_copy(rows[b], accum.at[idx_d.at[slot, b]], ssem.at[b],
                             add=True)

        def swait(b):
            pltpu.make_async_copy(rows[b], accum.at[idx_d.at[0, 0]],
                                  ssem.at[b]).wait()

        for b in range(NB):
            gstart(b, 0, b)

        @pl.loop(0, NROUND - 1)
        def _(k):
            q = k % 2
            pltpu.async_copy(srcs5.at[c, s, k + 1], idx_s.at[1 - q], isem)
            pltpu.async_copy(dsts4.at[s, k + 1], idx_d.at[1 - q], isem)
            for b in range(NB):
                gwait(b)
                compute(b)
                sstart(q, b)
            pltpu.make_async_copy(srcs5.at[0, 0, 0], idx_s.at[0],
                                  isem).wait()
            pltpu.make_async_copy(dsts4.at[0, 0], idx_d.at[0], isem).wait()
            for b in range(NB):
                swait(b)
                gstart((k + 1) * NB + b, 1 - q, b)

        ql = (NROUND - 1) % 2
        for b in range(NB):
            gwait(b)
            compute(b)
            sstart(ql, b)
        for b in range(NB):
            swait(b)

        plsc.subcore_barrier()
        pltpu.sync_copy(accum.at[pl.ds(s * RPT, RPT)],
                        out.at[pl.ds(s * RPT, RPT), pl.ds(col, HD)])

    return pl.kernel(
        body,
        out_type=jax.ShapeDtypeStruct((NPAD, D), jnp.float32),
        mesh=mesh,
        scratch_types=scratch,
        compiler_params=pltpu.CompilerParams(use_tc_tiling_on_sc=False),
    )


_segsum_plain = _make_segsum(False)
_segsum_edge = _make_segsum(True)


_BLK = 400


def _mlp(h, a, W1, b1, g, beta, W2, b2, post_relu):
    def body(h_ref, a_ref, W1_ref, b1_ref, g_ref, beta_ref, W2_ref, b2_ref,
             o_ref):
        t = h_ref[...] + a_ref[...]
        u = jnp.dot(t, W1_ref[...], preferred_element_type=jnp.float32)
        u = g_ref[...] * (u + b1_ref[...]) + beta_ref[...]
        u = jnp.maximum(u, 0.0)
        o = jnp.dot(u, W2_ref[...], preferred_element_type=jnp.float32)
        o = o + b2_ref[...]
        if post_relu:
            o = jnp.maximum(o, 0.0)
        o_ref[...] = o

    return pl.pallas_call(
        body,
        grid=(N // _BLK,),
        in_specs=[
            pl.BlockSpec((_BLK, D), lambda i: (i, 0)),
            pl.BlockSpec((_BLK, D), lambda i: (i, 0)),
            pl.BlockSpec((D, D), lambda i: (0, 0)),
            pl.BlockSpec((1, D), lambda i: (0, 0)),
            pl.BlockSpec((1, D), lambda i: (0, 0)),
            pl.BlockSpec((1, D), lambda i: (0, 0)),
            pl.BlockSpec((D, D), lambda i: (0, 0)),
            pl.BlockSpec((1, D), lambda i: (0, 0)),
        ],
        out_specs=pl.BlockSpec((_BLK, D), lambda i: (i, 0)),
        out_shape=jax.ShapeDtypeStruct((N, D), jnp.float32),
    )(h, a, W1, b1.reshape(1, D), g.reshape(1, D), beta.reshape(1, D), W2,
      b2.reshape(1, D))


_EBLK = 4000


def _edge_lin(edge_attr, We, bWe):
    def body(ea_ref, We_ref, b_ref, o_ref):
        e = (jnp.dot(ea_ref[...], We_ref[...],
                     preferred_element_type=jnp.float32) + b_ref[...])
        ep = e.reshape(_EBLK // 2, 2, D)
        o_ref[0] = jnp.concatenate([ep[:, 0, 0:HD], ep[:, 1, 0:HD]], axis=1)
        o_ref[1] = jnp.concatenate([ep[:, 0, HD:D], ep[:, 1, HD:D]], axis=1)

    return pl.pallas_call(
        body,
        grid=(E // _EBLK,),
        in_specs=[
            pl.BlockSpec((_EBLK, ED), lambda i: (i, 0)),
            pl.BlockSpec((ED, D), lambda i: (0, 0)),
            pl.BlockSpec((1, D), lambda i: (0, 0)),
        ],
        out_specs=pl.BlockSpec((NC, _EBLK // 2, D), lambda i: (0, i, 0)),
        out_shape=jax.ShapeDtypeStruct((NC, E // 2, D), jnp.float32),
    )(edge_attr, We, bWe.reshape(1, D))


def _pool(h, batch):
    nblk = N // _BLK

    def body(h_ref, b_ref, o_ref, s_ref, cnt_ref):
        i = pl.program_id(0)

        @pl.when(i == 0)
        def _():
            s_ref[...] = jnp.zeros_like(s_ref)
            cnt_ref[...] = jnp.zeros_like(cnt_ref)

        bb = b_ref[0, 0, :]
        iota = lax.broadcasted_iota(jnp.int32, (_BLK, B), 1)
        onehot = (bb[:, None] == iota).astype(jnp.float32)
        dn = (((0,), (0,)), ((), ()))
        s_ref[...] += lax.dot_general(onehot, h_ref[...], dn,
                                      preferred_element_type=jnp.float32)
        cnt_ref[...] += lax.dot_general(
            onehot, jnp.ones((_BLK, D), jnp.float32), dn,
            preferred_element_type=jnp.float32)

        @pl.when(i == nblk - 1)
        def _():
            o_ref[...] = s_ref[...] / jnp.maximum(cnt_ref[...], 1.0)

    return pl.pallas_call(
        body,
        grid=(nblk,),
        in_specs=[
            pl.BlockSpec((_BLK, D), lambda i: (i, 0)),
            pl.BlockSpec((1, 1, _BLK), lambda i: (i, 0, 0)),
        ],
        out_specs=pl.BlockSpec((B, D), lambda i: (0, 0)),
        out_shape=jax.ShapeDtypeStruct((B, D), jnp.float32),
        scratch_shapes=[pltpu.VMEM((B, D), jnp.float32),
                        pltpu.VMEM((B, D), jnp.float32)],
    )(h, batch.reshape(N // _BLK, 1, _BLK))


def kernel(x, edge_index, edge_attr, batch, W1_0, b1_0, g_0, beta_0, W2_0,
           b2_0, W1_1, b1_1, g_1, beta_1, W2_1, b2_1, W1_2, b1_2, g_2, beta_2,
           W2_2, b2_2, We, bWe):
    src2 = 2 * edge_index[0]
    srcs5 = jnp.stack([src2, src2 + 1]).reshape(NC, NS, NROUND, NB, C)
    dsts4 = edge_index[1].reshape(NS, NROUND, NB, C)
    zeros = jnp.zeros((NPAD, HD), jnp.float32)

    a = _segsum_plain(x.reshape(2 * N, HD), srcs5, dsts4, zeros)
    h = _mlp(x, a, W1_0, b1_0, g_0, beta_0, W2_0, b2_0, post_relu=True)

    e = _edge_lin(edge_attr, We, bWe)
    a = _segsum_edge(h.reshape(2 * N, HD), srcs5, dsts4, e, zeros)
    h = _mlp(h, a, W1_1, b1_1, g_1, beta_1, W2_1, b2_1, post_relu=True)

    a = _segsum_plain(h.reshape(2 * N, HD), srcs5, dsts4, zeros)
    h = _mlp(h, a, W1_2, b1_2, g_2, beta_2, W2_2, b2_2, post_relu=False)

    return _pool(h, batch)


# speedup vs baseline: 1.5076x; 1.5076x over previous
import jax
import jax.numpy as jnp
from jax import lax
from jax.experimental import pallas as pl
from jax.experimental.pallas import tpu as pltpu
from jax.experimental.pallas import tpu_sc as plsc

N = 10000
E = 320000
D = 128
ED = 16
B = 64

NC, NS = 2, 16
HD = D // NC
EPT = E // NS
C = 80
NCHUNK = EPT // C
NPAD = 10240
RPT = NPAD // NS

NB = 5
NROUND = NCHUNK // NB


def _make_segsum(with_edge):
    mesh = plsc.VectorSubcoreMesh(core_axis_name="c", subcore_axis_name="s")
    scratch = [
        pltpu.VMEM((2, NB, C), jnp.int32),
        pltpu.VMEM((2, NB, C), jnp.int32),
        [pltpu.VMEM((C, HD), jnp.float32) for _ in range(NB)],
    ]
    if with_edge:
        scratch.append([pltpu.VMEM((C, HD), jnp.float32) for _ in range(NB)])
    scratch += [
        pltpu.VMEM_SHARED((NPAD, HD), jnp.float32),
        pltpu.SemaphoreType.DMA((NB,)),
        pltpu.SemaphoreType.DMA((NB,)),
        pltpu.SemaphoreType.DMA,
    ]
    if with_edge:
        scratch.append(pltpu.SemaphoreType.DMA((NB,)))

    def body(table, srcs5, dsts4, *rest):
        if with_edge:
            (e_hbm, zeros, out, idx_s, idx_d, rows, ebuf, accum, gsem, ssem,
             isem, esem) = rest
        else:
            zeros, out, idx_s, idx_d, rows, accum, gsem, ssem, isem = rest
        c = lax.axis_index("c")
        s = lax.axis_index("s")
        col = c * HD
        pltpu.sync_copy(zeros.at[pl.ds(s * RPT, RPT)],
                        accum.at[pl.ds(s * RPT, RPT)])
        pltpu.sync_copy(srcs5.at[c, s, 0], idx_s.at[0])
        pltpu.sync_copy(dsts4.at[s, 0], idx_d.at[0])
        plsc.subcore_barrier()

        def gstart(chunk, slot, b):
            pltpu.async_copy(table.at[idx_s.at[slot, b]], rows[b],
                             gsem.at[b])
            if with_edge:
                pltpu.async_copy(
                    e_hbm.at[pl.ds(s * EPT + chunk * C, C), pl.ds(col, HD)],
                    ebuf[b], esem.at[b])

        def gwait(b):
            pltpu.make_async_copy(table.at[idx_s.at[0, 0]], rows[b],
                                  gsem.at[b]).wait()
            if with_edge:
                pltpu.make_async_copy(e_hbm.at[pl.ds(0, C), pl.ds(col, HD)],
                                      ebuf[b], esem.at[b]).wait()

        def compute(b):
            if with_edge:
                @plsc.parallel_loop(0, C, 1, unroll=8)
                def _(r):
                    for j in range(HD // 16):
                        sl = pl.ds(j * 16, 16)
                        rows[b][r, sl] = jnp.maximum(
                            rows[b][r, sl] + ebuf[b][r, sl], 0.0)

        def sstart(slot, b):
            pltpu.async_copy(rows[b], accum.at[idx_d.at[slot, b]], ssem.at[b],
                             add=True)

        def swait(b):
            pltpu.make_async_copy(rows[b], accum.at[idx_d.at[0, 0]],
                                  ssem.at[b]).wait()

        for b in range(NB):
            gstart(b, 0, b)

        @pl.loop(0, NROUND - 1)
        def _(k):
            q = k % 2
            pltpu.async_copy(srcs5.at[c, s, k + 1], idx_s.at[1 - q], isem)
            pltpu.async_copy(dsts4.at[s, k + 1], idx_d.at[1 - q], isem)
            for b in range(NB):
                gwait(b)
                compute(b)
                sstart(q, b)
            pltpu.make_async_copy(srcs5.at[0, 0, 0], idx_s.at[0],
                                  isem).wait()
            pltpu.make_async_copy(dsts4.at[0, 0], idx_d.at[0], isem).wait()
            for b in range(NB):
                swait(b)
                gstart((k + 1) * NB + b, 1 - q, b)

        ql = (NROUND - 1) % 2
        for b in range(NB):
            gwait(b)
            compute(b)
            sstart(ql, b)
        for b in range(NB):
            swait(b)

        plsc.subcore_barrier()
        pltpu.sync_copy(accum.at[pl.ds(s * RPT, RPT)],
                        out.at[pl.ds(s * RPT, RPT), pl.ds(col, HD)])

    return pl.kernel(
        body,
        out_type=jax.ShapeDtypeStruct((NPAD, D), jnp.float32),
        mesh=mesh,
        scratch_types=scratch,
        compiler_params=pltpu.CompilerParams(use_tc_tiling_on_sc=False),
    )


_segsum_plain = _make_segsum(False)
_segsum_edge = _make_segsum(True)


_BLK = 400


def _mlp(h, a, W1, b1, g, beta, W2, b2, post_relu):
    def body(h_ref, a_ref, W1_ref, b1_ref, g_ref, beta_ref, W2_ref, b2_ref,
             o_ref):
        t = h_ref[...] + a_ref[...]
        u = jnp.dot(t, W1_ref[...], preferred_element_type=jnp.float32)
        u = g_ref[...] * (u + b1_ref[...]) + beta_ref[...]
        u = jnp.maximum(u, 0.0)
        o = jnp.dot(u, W2_ref[...], preferred_element_type=jnp.float32)
        o = o + b2_ref[...]
        if post_relu:
            o = jnp.maximum(o, 0.0)
        o_ref[...] = o

    return pl.pallas_call(
        body,
        grid=(N // _BLK,),
        in_specs=[
            pl.BlockSpec((_BLK, D), lambda i: (i, 0)),
            pl.BlockSpec((_BLK, D), lambda i: (i, 0)),
            pl.BlockSpec((D, D), lambda i: (0, 0)),
            pl.BlockSpec((1, D), lambda i: (0, 0)),
            pl.BlockSpec((1, D), lambda i: (0, 0)),
            pl.BlockSpec((1, D), lambda i: (0, 0)),
            pl.BlockSpec((D, D), lambda i: (0, 0)),
            pl.BlockSpec((1, D), lambda i: (0, 0)),
        ],
        out_specs=pl.BlockSpec((_BLK, D), lambda i: (i, 0)),
        out_shape=jax.ShapeDtypeStruct((N, D), jnp.float32),
    )(h, a, W1, b1.reshape(1, D), g.reshape(1, D), beta.reshape(1, D), W2,
      b2.reshape(1, D))


_EBLK = 4000


def _edge_lin(edge_attr, We, bWe):
    def body(ea_ref, We_ref, b_ref, o_ref):
        o_ref[...] = (jnp.dot(ea_ref[...], We_ref[...],
                              preferred_element_type=jnp.float32)
                      + b_ref[...])

    return pl.pallas_call(
        body,
        grid=(E // _EBLK,),
        in_specs=[
            pl.BlockSpec((_EBLK, ED), lambda i: (i, 0)),
            pl.BlockSpec((ED, D), lambda i: (0, 0)),
            pl.BlockSpec((1, D), lambda i: (0, 0)),
        ],
        out_specs=pl.BlockSpec((_EBLK, D), lambda i: (i, 0)),
        out_shape=jax.ShapeDtypeStruct((E, D), jnp.float32),
    )(edge_attr, We, bWe.reshape(1, D))


def _pool(h, batch):
    nblk = N // _BLK

    def body(h_ref, b_ref, o_ref, s_ref, cnt_ref):
        i = pl.program_id(0)

        @pl.when(i == 0)
        def _():
            s_ref[...] = jnp.zeros_like(s_ref)
            cnt_ref[...] = jnp.zeros_like(cnt_ref)

        bb = b_ref[0, 0, :]
        iota = lax.broadcasted_iota(jnp.int32, (_BLK, B), 1)
        onehot = (bb[:, None] == iota).astype(jnp.float32)
        dn = (((0,), (0,)), ((), ()))
        s_ref[...] += lax.dot_general(onehot, h_ref[...], dn,
                                      preferred_element_type=jnp.float32)
        cnt_ref[...] += lax.dot_general(
            onehot, jnp.ones((_BLK, D), jnp.float32), dn,
            preferred_element_type=jnp.float32)

        @pl.when(i == nblk - 1)
        def _():
            o_ref[...] = s_ref[...] / jnp.maximum(cnt_ref[...], 1.0)

    return pl.pallas_call(
        body,
        grid=(nblk,),
        in_specs=[
            pl.BlockSpec((_BLK, D), lambda i: (i, 0)),
            pl.BlockSpec((1, 1, _BLK), lambda i: (i, 0, 0)),
        ],
        out_specs=pl.BlockSpec((B, D), lambda i: (0, 0)),
        out_shape=jax.ShapeDtypeStruct((B, D), jnp.float32),
        scratch_shapes=[pltpu.VMEM((B, D), jnp.float32),
                        pltpu.VMEM((B, D), jnp.float32)],
    )(h, batch.reshape(N // _BLK, 1, _BLK))


def kernel(x, edge_index, edge_attr, batch, W1_0, b1_0, g_0, beta_0, W2_0,
           b2_0, W1_1, b1_1, g_1, beta_1, W2_1, b2_1, W1_2, b1_2, g_2, beta_2,
           W2_2, b2_2, We, bWe):
    src2 = 2 * edge_index[0]
    srcs5 = jnp.stack([src2, src2 + 1]).reshape(NC, NS, NROUND, NB, C)
    dsts4 = edge_index[1].reshape(NS, NROUND, NB, C)
    zeros = jnp.zeros((NPAD, HD), jnp.float32)

    a = _segsum_plain(x.reshape(2 * N, HD), srcs5, dsts4, zeros)
    h = _mlp(x, a, W1_0, b1_0, g_0, beta_0, W2_0, b2_0, post_relu=True)

    e = _edge_lin(edge_attr, We, bWe)
    a = _segsum_edge(h.reshape(2 * N, HD), srcs5, dsts4, e, zeros)
    h = _mlp(h, a, W1_1, b1_1, g_1, beta_1, W2_1, b2_1, post_relu=True)

    a = _segsum_plain(h.reshape(2 * N, HD), srcs5, dsts4, zeros)
    h = _mlp(h, a, W1_2, b1_2, g_2, beta_2, W2_2, b2_2, post_relu=False)

    return _pool(h, batch)


# speedup vs baseline: 1.6286x; 1.0802x over previous
import jax
import jax.numpy as jnp
from jax import lax
from jax.experimental import pallas as pl
from jax.experimental.pallas import tpu as pltpu
from jax.experimental.pallas import tpu_sc as plsc

N = 10000
E = 320000
D = 128
ED = 16
B = 64

NC, NS = 2, 16
HD = D // NC
EPT = E // NS
C = 80
NCHUNK = EPT // C
NPAD = 10240
RPT = NPAD // NS

NB = 5
NROUND = NCHUNK // NB


def _make_segsum(with_edge):
    mesh = plsc.VectorSubcoreMesh(core_axis_name="c", subcore_axis_name="s")
    scratch = [
        pltpu.VMEM((2, NB, C), jnp.int32),
        pltpu.VMEM((2, NB, C), jnp.int32),
        [pltpu.VMEM((C, HD), jnp.float32) for _ in range(NB)],
    ]
    if with_edge:
        scratch.append([pltpu.VMEM((C, HD), jnp.float32) for _ in range(NB)])
    scratch += [
        pltpu.VMEM_SHARED((NPAD, HD), jnp.float32),
        pltpu.SemaphoreType.DMA((NB,)),
        pltpu.SemaphoreType.DMA((NB,)),
        pltpu.SemaphoreType.DMA,
    ]
    if with_edge:
        scratch.append(pltpu.SemaphoreType.DMA((NB,)))

    def body(table, srcs5, dsts4, *rest):
        if with_edge:
            (e_hbm, zeros, out, idx_s, idx_d, rows, ebuf, accum, gsem, ssem,
             isem, esem) = rest
        else:
            zeros, out, idx_s, idx_d, rows, accum, gsem, ssem, isem = rest
        c = lax.axis_index("c")
        s = lax.axis_index("s")
        col = c * HD
        pltpu.sync_copy(zeros.at[pl.ds(s * RPT, RPT)],
                        accum.at[pl.ds(s * RPT, RPT)])
        pltpu.sync_copy(srcs5.at[c, s, 0], idx_s.at[0])
        pltpu.sync_copy(dsts4.at[s, 0], idx_d.at[0])
        plsc.subcore_barrier()

        def gstart(chunk, slot, b):
            pltpu.async_copy(table.at[idx_s.at[slot, b]], rows[b],
                             gsem.at[b])
            if with_edge:
                pltpu.async_copy(
                    e_hbm.at[pl.ds(s * EPT + chunk * C, C), pl.ds(col, HD)],
                    ebuf[b], esem.at[b])

        def gwait(b):
            pltpu.make_async_copy(table.at[idx_s.at[0, 0]], rows[b],
                                  gsem.at[b]).wait()
            if with_edge:
                pltpu.make_async_copy(e_hbm.at[pl.ds(0, C), pl.ds(col, HD)],
                                      ebuf[b], esem.at[b]).wait()

        def compute(b):
            if with_edge:
                @plsc.parallel_loop(0, C, 1, unroll=8)
                def _(r):
                    for j in range(HD // 16):
                        sl = pl.ds(j * 16, 16)
                        rows[b][r, sl] = jnp.maximum(
                            rows[b][r, sl] + ebuf[b][r, sl], 0.0)

        def sstart(slot, b):
            pltpu.async_copy(rows[b], accum.at[idx_d.at[slot, b]], ssem.at[b],
                             add=True)

        def swait(b):
            pltpu.make_async_copy(rows[b], accum.at[idx_d.at[0, 0]],
                                  ssem.at[b]).wait()

        for b in range(NB):
            gstart(b, 0, b)

        @pl.loop(0, NROUND - 1)
        def _(k):
            q = k % 2
            pltpu.async_copy(srcs5.at[c, s, k + 1], idx_s.at[1 - q], isem)
            pltpu.async_copy(dsts4.at[s, k + 1], idx_d.at[1 - q], isem)
            for b in range(NB):
                gwait(b)
                compute(b)
                sstart(q, b)
            pltpu.make_async_copy(srcs5.at[0, 0, 0], idx_s.at[0],
                                  isem).wait()
            pltpu.make_async_copy(dsts4.at[0, 0], idx_d.at[0], isem).wait()
            for b in range(NB):
                swait(b)
                gstart((k + 1) * NB + b, 1 - q, b)

        ql = (NROUND - 1) % 2
        for b in range(NB):
            gwait(b)
            compute(b)
            sstart(ql, b)
        for b in range(NB):
            swait(b)

        plsc.subcore_barrier()
        pltpu.sync_copy(accum.at[pl.ds(s * RPT, RPT)],
                        out.at[pl.ds(s * RPT, RPT), pl.ds(col, HD)])

    return pl.kernel(
        body,
        out_type=jax.ShapeDtypeStruct((NPAD, D), jnp.float32),
        mesh=mesh,
        scratch_types=scratch,
        compiler_params=pltpu.CompilerParams(use_tc_tiling_on_sc=False),
    )


_segsum_plain = _make_segsum(False)
_segsum_edge = _make_segsum(True)


_BLK = 2000


def _mlp(h, a, W1, b1, g, beta, W2, b2, post_relu):
    def body(h_ref, a_ref, W1_ref, b1_ref, g_ref, beta_ref, W2_ref, b2_ref,
             o_ref):
        t = h_ref[...] + a_ref[...]
        u = jnp.dot(t, W1_ref[...], preferred_element_type=jnp.float32)
        u = g_ref[...] * (u + b1_ref[...]) + beta_ref[...]
        u = jnp.maximum(u, 0.0)
        o = jnp.dot(u, W2_ref[...], preferred_element_type=jnp.float32)
        o = o + b2_ref[...]
        if post_relu:
            o = jnp.maximum(o, 0.0)
        o_ref[...] = o

    return pl.pallas_call(
        body,
        grid=(N // _BLK,),
        in_specs=[
            pl.BlockSpec((_BLK, D), lambda i: (i, 0)),
            pl.BlockSpec((_BLK, D), lambda i: (i, 0)),
            pl.BlockSpec((D, D), lambda i: (0, 0)),
            pl.BlockSpec((1, D), lambda i: (0, 0)),
            pl.BlockSpec((1, D), lambda i: (0, 0)),
            pl.BlockSpec((1, D), lambda i: (0, 0)),
            pl.BlockSpec((D, D), lambda i: (0, 0)),
            pl.BlockSpec((1, D), lambda i: (0, 0)),
        ],
        out_specs=pl.BlockSpec((_BLK, D), lambda i: (i, 0)),
        out_shape=jax.ShapeDtypeStruct((N, D), jnp.float32),
    )(h, a, W1, b1.reshape(1, D), g.reshape(1, D), beta.reshape(1, D), W2,
      b2.reshape(1, D))


_EBLK = 4000


def _edge_lin(edge_attr, We, bWe):
    def body(ea_ref, We_ref, b_ref, o_ref):
        o_ref[...] = (jnp.dot(ea_ref[...], We_ref[...],
                              preferred_element_type=jnp.float32)
                      + b_ref[...])

    return pl.pallas_call(
        body,
        grid=(E // _EBLK,),
        in_specs=[
            pl.BlockSpec((_EBLK, ED), lambda i: (i, 0)),
            pl.BlockSpec((ED, D), lambda i: (0, 0)),
            pl.BlockSpec((1, D), lambda i: (0, 0)),
        ],
        out_specs=pl.BlockSpec((_EBLK, D), lambda i: (i, 0)),
        out_shape=jax.ShapeDtypeStruct((E, D), jnp.float32),
    )(edge_attr, We, bWe.reshape(1, D))


def _mlp_pool(h, a, W1, b1, g, beta, W2, b2, batch):
    nblk = N // _BLK

    def body(h_ref, a_ref, W1_ref, b1_ref, g_ref, beta_ref, W2_ref, b2_ref,
             b_ref, o_ref, s_ref, cnt_ref):
        i = pl.program_id(0)

        @pl.when(i == 0)
        def _():
            s_ref[...] = jnp.zeros_like(s_ref)
            cnt_ref[...] = jnp.zeros_like(cnt_ref)

        t = h_ref[...] + a_ref[...]
        u = jnp.dot(t, W1_ref[...], preferred_element_type=jnp.float32)
        u = g_ref[...] * (u + b1_ref[...]) + beta_ref[...]
        u = jnp.maximum(u, 0.0)
        o = jnp.dot(u, W2_ref[...], preferred_element_type=jnp.float32)
        o = o + b2_ref[...]

        bb = b_ref[0, 0, :]
        iota = lax.broadcasted_iota(jnp.int32, (_BLK, B), 1)
        onehot = (bb[:, None] == iota).astype(jnp.float32)
        dn = (((0,), (0,)), ((), ()))
        s_ref[...] += lax.dot_general(onehot, o, dn,
                                      preferred_element_type=jnp.float32)
        cnt_ref[...] += lax.dot_general(
            onehot, jnp.ones((_BLK, D), jnp.float32), dn,
            preferred_element_type=jnp.float32)

        @pl.when(i == nblk - 1)
        def _():
            o_ref[...] = s_ref[...] / jnp.maximum(cnt_ref[...], 1.0)

    return pl.pallas_call(
        body,
        grid=(nblk,),
        in_specs=[
            pl.BlockSpec((_BLK, D), lambda i: (i, 0)),
            pl.BlockSpec((_BLK, D), lambda i: (i, 0)),
            pl.BlockSpec((D, D), lambda i: (0, 0)),
            pl.BlockSpec((1, D), lambda i: (0, 0)),
            pl.BlockSpec((1, D), lambda i: (0, 0)),
            pl.BlockSpec((1, D), lambda i: (0, 0)),
            pl.BlockSpec((D, D), lambda i: (0, 0)),
            pl.BlockSpec((1, D), lambda i: (0, 0)),
            pl.BlockSpec((1, 1, _BLK), lambda i: (i, 0, 0)),
        ],
        out_specs=pl.BlockSpec((B, D), lambda i: (0, 0)),
        out_shape=jax.ShapeDtypeStruct((B, D), jnp.float32),
        scratch_shapes=[pltpu.VMEM((B, D), jnp.float32),
                        pltpu.VMEM((B, D), jnp.float32)],
    )(h, a, W1, b1.reshape(1, D), g.reshape(1, D), beta.reshape(1, D), W2,
      b2.reshape(1, D), batch.reshape(N // _BLK, 1, _BLK))


def kernel(x, edge_index, edge_attr, batch, W1_0, b1_0, g_0, beta_0, W2_0,
           b2_0, W1_1, b1_1, g_1, beta_1, W2_1, b2_1, W1_2, b1_2, g_2, beta_2,
           W2_2, b2_2, We, bWe):
    src2 = 2 * edge_index[0]
    srcs5 = jnp.stack([src2, src2 + 1]).reshape(NC, NS, NROUND, NB, C)
    dsts4 = edge_index[1].reshape(NS, NROUND, NB, C)
    zeros = jnp.zeros((NPAD, HD), jnp.float32)

    a = _segsum_plain(x.reshape(2 * N, HD), srcs5, dsts4, zeros)
    h = _mlp(x, a, W1_0, b1_0, g_0, beta_0, W2_0, b2_0, post_relu=True)

    e = _edge_lin(edge_attr, We, bWe)
    a = _segsum_edge(h.reshape(2 * N, HD), srcs5, dsts4, e, zeros)
    h = _mlp(h, a, W1_1, b1_1, g_1, beta_1, W2_1, b2_1, post_relu=True)

    a = _segsum_plain(h.reshape(2 * N, HD), srcs5, dsts4, zeros)
    return _mlp_pool(h, a, W1_2, b1_2, g_2, beta_2, W2_2, b2_2, batch)


# speedup vs baseline: 1.6532x; 1.0151x over previous
import jax
import jax.numpy as jnp
from jax import lax
from jax.experimental import pallas as pl
from jax.experimental.pallas import tpu as pltpu
from jax.experimental.pallas import tpu_sc as plsc

N = 10000
E = 320000
D = 128
ED = 16
B = 64

NC, NS = 2, 16
HD = D // NC
EPT = E // NS
C = 80
NCHUNK = EPT // C
NPAD = 10240
RPT = NPAD // NS

NB = 5
NROUND = NCHUNK // NB

C1 = 40
NCHUNK1 = EPT // C1
NROUND1 = NCHUNK1 // NB


def _make_segsum(with_edge):
    mesh = plsc.VectorSubcoreMesh(core_axis_name="c", subcore_axis_name="s")
    scratch = [
        pltpu.VMEM((2, NB, C), jnp.int32),
        pltpu.VMEM((2, NB, C), jnp.int32),
        [pltpu.VMEM((C, HD), jnp.float32) for _ in range(NB)],
    ]
    if with_edge:
        scratch.append([pltpu.VMEM((C, HD), jnp.float32) for _ in range(NB)])
    scratch += [
        pltpu.VMEM_SHARED((NPAD, HD), jnp.float32),
        pltpu.SemaphoreType.DMA((NB,)),
        pltpu.SemaphoreType.DMA((NB,)),
        pltpu.SemaphoreType.DMA,
    ]
    if with_edge:
        scratch.append(pltpu.SemaphoreType.DMA((NB,)))

    def body(table, srcs5, dsts4, *rest):
        if with_edge:
            (e_hbm, zeros, out, idx_s, idx_d, rows, ebuf, accum, gsem, ssem,
             isem, esem) = rest
        else:
            zeros, out, idx_s, idx_d, rows, accum, gsem, ssem, isem = rest
        c = lax.axis_index("c")
        s = lax.axis_index("s")
        col = c * HD
        pltpu.sync_copy(zeros.at[pl.ds(s * RPT, RPT)],
                        accum.at[pl.ds(s * RPT, RPT)])
        pltpu.sync_copy(srcs5.at[c, s, 0], idx_s.at[0])
        pltpu.sync_copy(dsts4.at[s, 0], idx_d.at[0])
        plsc.subcore_barrier()

        def gstart(chunk, slot, b):
            pltpu.async_copy(table.at[idx_s.at[slot, b]], rows[b],
                             gsem.at[b])
            if with_edge:
                pltpu.async_copy(
                    e_hbm.at[pl.ds(s * EPT + chunk * C, C), pl.ds(col, HD)],
                    ebuf[b], esem.at[b])

        def gwait(b):
            pltpu.make_async_copy(table.at[idx_s.at[0, 0]], rows[b],
                                  gsem.at[b]).wait()
            if with_edge:
                pltpu.make_async_copy(e_hbm.at[pl.ds(0, C), pl.ds(col, HD)],
                                      ebuf[b], esem.at[b]).wait()

        def compute(b):
            if with_edge:
                @plsc.parallel_loop(0, C, 1, unroll=8)
                def _(r):
                    for j in range(HD // 16):
                        sl = pl.ds(j * 16, 16)
                        rows[b][r, sl] = jnp.maximum(
                            rows[b][r, sl] + ebuf[b][r, sl], 0.0)

        def sstart(slot, b):
            pltpu.async_copy(rows[b], accum.at[idx_d.at[slot, b]], ssem.at[b],
                             add=True)

        def swait(b):
            pltpu.make_async_copy(rows[b], accum.at[idx_d.at[0, 0]],
                                  ssem.at[b]).wait()

        for b in range(NB):
            gstart(b, 0, b)

        @pl.loop(0, NROUND - 1)
        def _(k):
            q = k % 2
            pltpu.async_copy(srcs5.at[c, s, k + 1], idx_s.at[1 - q], isem)
            pltpu.async_copy(dsts4.at[s, k + 1], idx_d.at[1 - q], isem)
            for b in range(NB):
                gwait(b)
                compute(b)
                sstart(q, b)
            pltpu.make_async_copy(srcs5.at[0, 0, 0], idx_s.at[0],
                                  isem).wait()
            pltpu.make_async_copy(dsts4.at[0, 0], idx_d.at[0], isem).wait()
            for b in range(NB):
                swait(b)
                gstart((k + 1) * NB + b, 1 - q, b)

        ql = (NROUND - 1) % 2
        for b in range(NB):
            gwait(b)
            compute(b)
            sstart(ql, b)
        for b in range(NB):
            swait(b)

        plsc.subcore_barrier()
        pltpu.sync_copy(accum.at[pl.ds(s * RPT, RPT)],
                        out.at[pl.ds(s * RPT, RPT), pl.ds(col, HD)])

    return pl.kernel(
        body,
        out_type=jax.ShapeDtypeStruct((NPAD, D), jnp.float32),
        mesh=mesh,
        scratch_types=scratch,
        compiler_params=pltpu.CompilerParams(use_tc_tiling_on_sc=False),
    )


_segsum_plain = _make_segsum(False)


def _make_segsum_edge():
    mesh = plsc.VectorSubcoreMesh(core_axis_name="c", subcore_axis_name="s")
    scratch = [
        pltpu.VMEM((2, NB, C1), jnp.int32),
        pltpu.VMEM((2, NB, C1), jnp.int32),
        [pltpu.VMEM((C1, HD), jnp.float32) for _ in range(NB)],
        [pltpu.VMEM((C1, HD), jnp.float32) for _ in range(NB)],
        pltpu.VMEM_SHARED((N, HD), jnp.float32),
        pltpu.VMEM_SHARED((NPAD, HD), jnp.float32),
        pltpu.SemaphoreType.DMA((NB,)),
        pltpu.SemaphoreType.DMA((NB,)),
        pltpu.SemaphoreType.DMA,
        pltpu.SemaphoreType.DMA((NB,)),
    ]

    def body(table, srcs4, dsts4, e_hbm, zeros, out, idx_s, idx_d, rows,
             ebuf, spm, accum, gsem, ssem, isem, esem):
        c = lax.axis_index("c")
        s = lax.axis_index("s")
        col = c * HD
        off = jnp.minimum(s * RPT, N - RPT)
        pltpu.sync_copy(table.at[pl.ds(off, RPT), pl.ds(col, HD)],
                        spm.at[pl.ds(off, RPT)])
        pltpu.sync_copy(zeros.at[pl.ds(s * RPT, RPT)],
                        accum.at[pl.ds(s * RPT, RPT)])
        pltpu.sync_copy(srcs4.at[s, 0], idx_s.at[0])
        pltpu.sync_copy(dsts4.at[s, 0], idx_d.at[0])
        plsc.subcore_barrier()

        def gstart(chunk, slot, b):
            pltpu.async_copy(spm.at[idx_s.at[slot, b]], rows[b], gsem.at[b])
            pltpu.async_copy(
                e_hbm.at[pl.ds(s * EPT + chunk * C1, C1), pl.ds(col, HD)],
                ebuf[b], esem.at[b])

        def gwait(b):
            pltpu.make_async_copy(spm.at[idx_s.at[0, 0]], rows[b],
                                  gsem.at[b]).wait()
            pltpu.make_async_copy(e_hbm.at[pl.ds(0, C1), pl.ds(col, HD)],
                                  ebuf[b], esem.at[b]).wait()

        def compute(b):
            @plsc.parallel_loop(0, C1, 1, unroll=8)
            def _(r):
                for j in range(HD // 16):
                    sl = pl.ds(j * 16, 16)
                    rows[b][r, sl] = jnp.maximum(
                        rows[b][r, sl] + ebuf[b][r, sl], 0.0)

        def sstart(slot, b):
            pltpu.async_copy(rows[b], accum.at[idx_d.at[slot, b]],
                             ssem.at[b], add=True)

        def swait(b):
            pltpu.make_async_copy(rows[b], accum.at[idx_d.at[0, 0]],
                                  ssem.at[b]).wait()

        for b in range(NB):
            gstart(b, 0, b)

        @pl.loop(0, NROUND1 - 1)
        def _(k):
            q = k % 2
            pltpu.async_copy(srcs4.at[s, k + 1], idx_s.at[1 - q], isem)
            pltpu.async_copy(dsts4.at[s, k + 1], idx_d.at[1 - q], isem)
            for b in range(NB):
                gwait(b)
                compute(b)
                sstart(q, b)
            pltpu.make_async_copy(srcs4.at[0, 0], idx_s.at[0], isem).wait()
            pltpu.make_async_copy(dsts4.at[0, 0], idx_d.at[0], isem).wait()
            for b in range(NB):
                swait(b)
                gstart((k + 1) * NB + b, 1 - q, b)

        ql = (NROUND1 - 1) % 2
        for b in range(NB):
            gwait(b)
            compute(b)
            sstart(ql, b)
        for b in range(NB):
            swait(b)

        plsc.subcore_barrier()
        pltpu.sync_copy(accum.at[pl.ds(s * RPT, RPT)],
                        out.at[pl.ds(s * RPT, RPT), pl.ds(col, HD)])

    return pl.kernel(
        body,
        out_type=jax.ShapeDtypeStruct((NPAD, D), jnp.float32),
        mesh=mesh,
        scratch_types=scratch,
        compiler_params=pltpu.CompilerParams(use_tc_tiling_on_sc=False),
    )


_segsum_edge = _make_segsum_edge()


_BLK = 2000


def _mlp(h, a, W1, b1, g, beta, W2, b2, post_relu):
    def body(h_ref, a_ref, W1_ref, b1_ref, g_ref, beta_ref, W2_ref, b2_ref,
             o_ref):
        t = h_ref[...] + a_ref[...]
        u = jnp.dot(t, W1_ref[...], preferred_element_type=jnp.float32)
        u = g_ref[...] * (u + b1_ref[...]) + beta_ref[...]
        u = jnp.maximum(u, 0.0)
        o = jnp.dot(u, W2_ref[...], preferred_element_type=jnp.float32)
        o = o + b2_ref[...]
        if post_relu:
            o = jnp.maximum(o, 0.0)
        o_ref[...] = o

    return pl.pallas_call(
        body,
        grid=(N // _BLK,),
        in_specs=[
            pl.BlockSpec((_BLK, D), lambda i: (i, 0)),
            pl.BlockSpec((_BLK, D), lambda i: (i, 0)),
            pl.BlockSpec((D, D), lambda i: (0, 0)),
            pl.BlockSpec((1, D), lambda i: (0, 0)),
            pl.BlockSpec((1, D), lambda i: (0, 0)),
            pl.BlockSpec((1, D), lambda i: (0, 0)),
            pl.BlockSpec((D, D), lambda i: (0, 0)),
            pl.BlockSpec((1, D), lambda i: (0, 0)),
        ],
        out_specs=pl.BlockSpec((_BLK, D), lambda i: (i, 0)),
        out_shape=jax.ShapeDtypeStruct((N, D), jnp.float32),
    )(h, a, W1, b1.reshape(1, D), g.reshape(1, D), beta.reshape(1, D), W2,
      b2.reshape(1, D))


_EBLK = 4000


def _edge_lin(edge_attr, We, bWe):
    def body(ea_ref, We_ref, b_ref, o_ref):
        o_ref[...] = (jnp.dot(ea_ref[...], We_ref[...],
                              preferred_element_type=jnp.float32)
                      + b_ref[...])

    return pl.pallas_call(
        body,
        grid=(E // _EBLK,),
        in_specs=[
            pl.BlockSpec((_EBLK, ED), lambda i: (i, 0)),
            pl.BlockSpec((ED, D), lambda i: (0, 0)),
            pl.BlockSpec((1, D), lambda i: (0, 0)),
        ],
        out_specs=pl.BlockSpec((_EBLK, D), lambda i: (i, 0)),
        out_shape=jax.ShapeDtypeStruct((E, D), jnp.float32),
    )(edge_attr, We, bWe.reshape(1, D))


def _mlp_pool(h, a, W1, b1, g, beta, W2, b2, batch):
    nblk = N // _BLK

    def body(h_ref, a_ref, W1_ref, b1_ref, g_ref, beta_ref, W2_ref, b2_ref,
             b_ref, o_ref, s_ref, cnt_ref):
        i = pl.program_id(0)

        @pl.when(i == 0)
        def _():
            s_ref[...] = jnp.zeros_like(s_ref)
            cnt_ref[...] = jnp.zeros_like(cnt_ref)

        t = h_ref[...] + a_ref[...]
        u = jnp.dot(t, W1_ref[...], preferred_element_type=jnp.float32)
        u = g_ref[...] * (u + b1_ref[...]) + beta_ref[...]
        u = jnp.maximum(u, 0.0)
        o = jnp.dot(u, W2_ref[...], preferred_element_type=jnp.float32)
        o = o + b2_ref[...]

        bb = b_ref[0, 0, :]
        iota = lax.broadcasted_iota(jnp.int32, (_BLK, B), 1)
        onehot = (bb[:, None] == iota).astype(jnp.float32)
        dn = (((0,), (0,)), ((), ()))
        s_ref[...] += lax.dot_general(onehot, o, dn,
                                      preferred_element_type=jnp.float32)
        cnt_ref[...] += lax.dot_general(
            onehot, jnp.ones((_BLK, D), jnp.float32), dn,
            preferred_element_type=jnp.float32)

        @pl.when(i == nblk - 1)
        def _():
            o_ref[...] = s_ref[...] / jnp.maximum(cnt_ref[...], 1.0)

    return pl.pallas_call(
        body,
        grid=(nblk,),
        in_specs=[
            pl.BlockSpec((_BLK, D), lambda i: (i, 0)),
            pl.BlockSpec((_BLK, D), lambda i: (i, 0)),
            pl.BlockSpec((D, D), lambda i: (0, 0)),
            pl.BlockSpec((1, D), lambda i: (0, 0)),
            pl.BlockSpec((1, D), lambda i: (0, 0)),
            pl.BlockSpec((1, D), lambda i: (0, 0)),
            pl.BlockSpec((D, D), lambda i: (0, 0)),
            pl.BlockSpec((1, D), lambda i: (0, 0)),
            pl.BlockSpec((1, 1, _BLK), lambda i: (i, 0, 0)),
        ],
        out_specs=pl.BlockSpec((B, D), lambda i: (0, 0)),
        out_shape=jax.ShapeDtypeStruct((B, D), jnp.float32),
        scratch_shapes=[pltpu.VMEM((B, D), jnp.float32),
                        pltpu.VMEM((B, D), jnp.float32)],
    )(h, a, W1, b1.reshape(1, D), g.reshape(1, D), beta.reshape(1, D), W2,
      b2.reshape(1, D), batch.reshape(N // _BLK, 1, _BLK))


def kernel(x, edge_index, edge_attr, batch, W1_0, b1_0, g_0, beta_0, W2_0,
           b2_0, W1_1, b1_1, g_1, beta_1, W2_1, b2_1, W1_2, b1_2, g_2, beta_2,
           W2_2, b2_2, We, bWe):
    src2 = 2 * edge_index[0]
    srcs5 = jnp.stack([src2, src2 + 1]).reshape(NC, NS, NROUND, NB, C)
    dsts4 = edge_index[1].reshape(NS, NROUND, NB, C)
    zeros = jnp.zeros((NPAD, HD), jnp.float32)

    a = _segsum_plain(x.reshape(2 * N, HD), srcs5, dsts4, zeros)
    h = _mlp(x, a, W1_0, b1_0, g_0, beta_0, W2_0, b2_0, post_relu=True)

    e = _edge_lin(edge_attr, We, bWe)
    srcs4e = edge_index[0].reshape(NS, NROUND1, NB, C1)
    dsts4e = edge_index[1].reshape(NS, NROUND1, NB, C1)
    a = _segsum_edge(h, srcs4e, dsts4e, e, zeros)
    h = _mlp(h, a, W1_1, b1_1, g_1, beta_1, W2_1, b2_1, post_relu=True)

    a = _segsum_plain(h.reshape(2 * N, HD), srcs5, dsts4, zeros)
    return _mlp_pool(h, a, W1_2, b1_2, g_2, beta_2, W2_2, b2_2, batch)
